# Initial kernel scaffold; baseline (speedup 1.0000x reference)
import jax
import jax.numpy as jnp
from jax.experimental import pallas as pl


def kernel(x, field_start_idx_array, starter_idx_array, table):
    raise NotImplementedError("write your pallas kernel here")


# speedup vs baseline: 1.0909x; 1.0909x over previous
import functools

import jax
import jax.numpy as jnp
from jax import lax
from jax.experimental import pallas as pl
from jax.experimental.pallas import tpu as pltpu
from jax.experimental.pallas import tpu_sc as plsc

BATCH = 4096
NFIELD = 282
NPOS = 283
DIM = 64
FPAD = 288
NC = 2
NS = 16
NW = NC * NS
ROWS_PER_W = BATCH // NW
GCHUNKS = ((0, 128), (128, 128), (256, 32))
SLAB = NPOS + 6


def _body(x_hbm, starts_hbm, starter_hbm, table_hbm, out_hbm,
          starts_v, x_v, idx_v, rows_v, sem):
    wid = lax.axis_index("s") * NC + lax.axis_index("c")
    base = wid * ROWS_PER_W

    zero16 = jnp.zeros((16,), jnp.int32)
    x_v[pl.ds(FPAD - 16, 16)] = zero16
    starts_v[pl.ds(FPAD - 16, 16)] = zero16
    pltpu.sync_copy(starts_hbm, starts_v.at[pl.ds(0, NFIELD)])

    pltpu.sync_copy(starter_hbm, idx_v.at[pl.ds(0, 1)])
    pltpu.async_copy(table_hbm.at[idx_v.at[pl.ds(0, 1)]],
                     rows_v.at[pl.ds(0, 1)], sem).wait()

    def row_body(i, carry):
        b = base + i
        pltpu.sync_copy(x_hbm.at[b], x_v.at[pl.ds(0, NFIELD)])
        for v in range(FPAD // 16):
            sl = pl.ds(v * 16, 16)
            idx_v[sl] = x_v[sl] + starts_v[sl]
        copies = [
            pltpu.async_copy(table_hbm.at[idx_v.at[pl.ds(off, ln)]],
                             rows_v.at[pl.ds(1 + off, ln)], sem)
            for off, ln in GCHUNKS
        ]
        for cp in copies:
            cp.wait()
        pltpu.sync_copy(rows_v.at[pl.ds(0, NPOS)],
                        out_hbm.at[pl.ds(b * NPOS, NPOS)])
        return carry

    lax.fori_loop(0, ROWS_PER_W, row_body, 0)


@functools.partial(jax.jit, static_argnames=())
def _emb_lookup(x, starts, starter, table):
    mesh = plsc.VectorSubcoreMesh(core_axis_name="c", subcore_axis_name="s")
    run = functools.partial(
        pl.kernel,
        mesh=mesh,
        out_type=jax.ShapeDtypeStruct((BATCH * NPOS, DIM), jnp.float32),
        compiler_params=pltpu.CompilerParams(use_tc_tiling_on_sc=False),
        scratch_types=[
            pltpu.VMEM((FPAD,), jnp.int32),
            pltpu.VMEM((FPAD,), jnp.int32),
            pltpu.VMEM((FPAD,), jnp.int32),
            pltpu.VMEM((SLAB, DIM), jnp.float32),
            pltpu.SemaphoreType.DMA,
        ],
    )(_body)
    flat = run(x, starts, starter, table)
    return flat.reshape(BATCH, NPOS, DIM)


def kernel(x, field_start_idx_array, starter_idx_array, table):
    return _emb_lookup(x, field_start_idx_array, starter_idx_array, table)


# speedup vs baseline: 12.7855x; 11.7206x over previous
import functools

import jax
import jax.numpy as jnp
from jax import lax
from jax.experimental import pallas as pl
from jax.experimental.pallas import tpu as pltpu
from jax.experimental.pallas import tpu_sc as plsc

BATCH = 4096
NFIELD = 282
NPOS = 283
DIM = 64
FPAD = 288
NC = 2
NS = 16
NW = NC * NS
ROWS_PER_W = BATCH // NW
GCHUNKS = ((0, 128), (128, 128), (256, 32))
SLAB = NPOS + 6


def _body(x_hbm, starts_hbm, starter_hbm, table_hbm, out_hbm,
          table_sp, starts_v, x_blk, idx_a, idx_b, slab_a, slab_b,
          gsem, wsem_a, wsem_b):
    wid = lax.axis_index("s") * NC + lax.axis_index("c")
    base = wid * ROWS_PER_W

    @pl.when(lax.axis_index("s") == 0)
    def _():
        pltpu.sync_copy(table_hbm, table_sp)

    zero16 = jnp.zeros((16,), jnp.int32)
    starts_v[pl.ds(FPAD - 16, 16)] = zero16
    pltpu.sync_copy(starts_hbm, starts_v.at[pl.ds(0, NFIELD)])
    pltpu.sync_copy(x_hbm.at[pl.ds(base, ROWS_PER_W)], x_blk)
    plsc.subcore_barrier()

    pltpu.sync_copy(starter_hbm, idx_a.at[pl.ds(0, 1)])
    pltpu.async_copy(table_sp.at[idx_a.at[pl.ds(0, 1)]],
                     slab_a.at[pl.ds(0, 1)], gsem).wait()
    pltpu.async_copy(table_sp.at[idx_a.at[pl.ds(0, 1)]],
                     slab_b.at[pl.ds(0, 1)], gsem).wait()

    def fill(i, idx_v, slab_v):
        for v in range(FPAD // 16):
            sl = pl.ds(v * 16, 16)
            idx_v[sl] = x_blk[i, sl] + starts_v[sl]
        return [
            pltpu.async_copy(table_sp.at[idx_v.at[pl.ds(off, ln)]],
                             slab_v.at[pl.ds(1 + off, ln)], gsem)
            for off, ln in GCHUNKS
        ]

    def writeback(b, slab_v, wsem):
        return pltpu.async_copy(slab_v.at[pl.ds(0, NPOS)],
                                out_hbm.at[pl.ds(b * NPOS, NPOS)], wsem)

    def drain(slab_v, wsem):
        pltpu.make_async_copy(slab_v.at[pl.ds(0, NPOS)],
                              out_hbm.at[pl.ds(0, NPOS)], wsem).wait()

    def pair_body(i, carry):
        i0 = 2 * i
        i1 = i0 + 1

        @pl.when(i > 0)
        def _():
            drain(slab_a, wsem_a)
        cps_a = fill(i0, idx_a, slab_a)

        @pl.when(i > 0)
        def _():
            drain(slab_b, wsem_b)
        for cp in cps_a:
            cp.wait()
        writeback(base + i0, slab_a, wsem_a)

        cps_b = fill(i1, idx_b, slab_b)
        for cp in cps_b:
            cp.wait()
        writeback(base + i1, slab_b, wsem_b)
        return carry

    lax.fori_loop(0, ROWS_PER_W // 2, pair_body, 0)
    drain(slab_a, wsem_a)
    drain(slab_b, wsem_b)


@functools.partial(jax.jit, static_argnames=())
def _emb_lookup(x, starts, starter, table):
    mesh = plsc.VectorSubcoreMesh(core_axis_name="c", subcore_axis_name="s")
    run = functools.partial(
        pl.kernel,
        mesh=mesh,
        out_type=jax.ShapeDtypeStruct((BATCH * NPOS, DIM), jnp.float32),
        compiler_params=pltpu.CompilerParams(use_tc_tiling_on_sc=False),
        scratch_types=[
            pltpu.VMEM_SHARED((790, DIM), jnp.float32),
            pltpu.VMEM((FPAD,), jnp.int32),
            pltpu.VMEM((ROWS_PER_W, FPAD), jnp.int32),
            pltpu.VMEM((FPAD,), jnp.int32),
            pltpu.VMEM((FPAD,), jnp.int32),
            pltpu.VMEM((SLAB, DIM), jnp.float32),
            pltpu.VMEM((SLAB, DIM), jnp.float32),
            pltpu.SemaphoreType.DMA,
            pltpu.SemaphoreType.DMA,
            pltpu.SemaphoreType.DMA,
        ],
    )(_body)
    xp = jnp.pad(x, ((0, 0), (0, FPAD - NFIELD)))
    flat = run(xp, starts, starter, table)
    return flat.reshape(BATCH, NPOS, DIM)


def kernel(x, field_start_idx_array, starter_idx_array, table):
    return _emb_lookup(x, field_start_idx_array, starter_idx_array, table)
